# baseline (speedup 1.0000x reference)
import functools

import jax
import jax.numpy as jnp
from jax import lax
from jax.experimental import pallas as pl
from jax.experimental.pallas import tpu as pltpu
from jax.experimental.pallas import tpu_sc as plsc

N = 10000
E = 320000
F = 128
R = 16

ALPHA = 0.01
MU = 2.5

NC = 2
NS = 16
NW = NC * NS
EPW = E // NW
CHUNK = 80
NCHUNK = EPW // CHUNK
NPAD = 10240
ROWS_PER_S = NPAD // NS


def _smu(x):
    return ((1 + ALPHA) * x
            + (1 - ALPHA) * x * lax.erf(MU * (1 - ALPHA) * x)) / 2


def _h_body(x_ref, w_ref, o_ref):
    o_ref[...] = jnp.dot(x_ref[...], w_ref[...],
                         preferred_element_type=jnp.float32)


def _compute_h(x, w):
    blk = 2000
    return pl.pallas_call(
        _h_body,
        grid=(N // blk,),
        in_specs=[
            pl.BlockSpec((blk, F), lambda i: (i, 0)),
            pl.BlockSpec((F, F), lambda i: (0, 0)),
        ],
        out_specs=pl.BlockSpec((blk, F), lambda i: (i, 0)),
        out_shape=jax.ShapeDtypeStruct((N, F), jnp.float32),
    )(x, w)


def _wij_body(f_ref, w1_ref, b1_ref, w2_ref, b2_ref, o_ref):
    t = jnp.dot(f_ref[...], w1_ref[...], preferred_element_type=jnp.float32)
    t = _smu(t + b1_ref[...])
    w = jnp.dot(t, w2_ref[...], preferred_element_type=jnp.float32)
    o_ref[...] = w + b2_ref[...]


def _compute_wij(f_ij, w1, b1, w2, b2):
    blk = 4000
    return pl.pallas_call(
        _wij_body,
        grid=(E // blk,),
        in_specs=[
            pl.BlockSpec((blk, R), lambda i: (i, 0)),
            pl.BlockSpec((R, F), lambda i: (0, 0)),
            pl.BlockSpec((1, F), lambda i: (0, 0)),
            pl.BlockSpec((F, F), lambda i: (0, 0)),
            pl.BlockSpec((1, F), lambda i: (0, 0)),
        ],
        out_specs=pl.BlockSpec((blk, F), lambda i: (i, 0)),
        out_shape=jax.ShapeDtypeStruct((E, F), jnp.float32),
    )(f_ij, w1, b1, w2, b2)


def _mul_rows(xj, wij, rc_v, p):
    def row_body(r, c2):
        ro = pl.multiple_of((r % 8) * 16, 16)
        rcv = rc_v[p, r // 8, pl.ds(ro, 16)]
        for k in range(F // 16):
            sl = pl.ds(k * 16, 16)
            xj[r, sl] = xj[r, sl] * wij[r, sl] * rcv
        return c2
    lax.fori_loop(0, CHUNK, row_body, 0)


def _sc_body(h_hbm, wij_hbm, idx_hbm, rc_hbm, zeros_hbm, out_hbm,
             idx_v, rc_v, xj_a, wij_a, xj_b, wij_b, agg_sh,
             g_a, w_a, s_a, g_b, w_b, s_b):
    cid = lax.axis_index("c")
    sid = lax.axis_index("s")
    wid = sid * NC + cid
    base_e = wid * EPW

    pltpu.sync_copy(zeros_hbm.at[pl.ds(sid * ROWS_PER_S, ROWS_PER_S)],
                    agg_sh.at[pl.ds(sid * ROWS_PER_S, ROWS_PER_S)])
    plsc.subcore_barrier()

    def pair_body(i, carry):
        ta = 2 * i
        tb = 2 * i + 1
        offa = pl.multiple_of(base_e + ta * CHUNK, 16)
        offb = pl.multiple_of(base_e + tb * CHUNK, 16)
        pltpu.sync_copy(idx_hbm.at[wid, pl.ds(ta, 2)], idx_v)
        pltpu.sync_copy(rc_hbm.at[wid, pl.ds(ta, 2)], rc_v)
        cga = pltpu.async_copy(h_hbm.at[idx_v.at[0, 1]], xj_a, g_a)
        cwa = pltpu.async_copy(wij_hbm.at[pl.ds(offa, CHUNK)], wij_a, w_a)
        cgb = pltpu.async_copy(h_hbm.at[idx_v.at[1, 1]], xj_b, g_b)
        cwb = pltpu.async_copy(wij_hbm.at[pl.ds(offb, CHUNK)], wij_b, w_b)
        cga.wait()
        cwa.wait()
        _mul_rows(xj_a, wij_a, rc_v, 0)
        csa = pltpu.async_copy(xj_a, agg_sh.at[idx_v.at[0, 0]], s_a, add=True)
        cgb.wait()
        cwb.wait()
        _mul_rows(xj_b, wij_b, rc_v, 1)
        csb = pltpu.async_copy(xj_b, agg_sh.at[idx_v.at[1, 0]], s_b, add=True)
        csa.wait()
        csb.wait()
        return carry

    lax.fori_loop(0, NCHUNK // 2, pair_body, 0)

    tt = NCHUNK - 1
    offt = pl.multiple_of(base_e + tt * CHUNK, 16)
    pltpu.sync_copy(idx_hbm.at[wid, pl.ds(tt - 1, 2)], idx_v)
    pltpu.sync_copy(rc_hbm.at[wid, pl.ds(tt - 1, 2)], rc_v)
    cgt = pltpu.async_copy(h_hbm.at[idx_v.at[1, 1]], xj_a, g_a)
    cwt = pltpu.async_copy(wij_hbm.at[pl.ds(offt, CHUNK)], wij_a, w_a)
    cgt.wait()
    cwt.wait()
    _mul_rows(xj_a, wij_a, rc_v, 1)
    pltpu.async_copy(xj_a, agg_sh.at[idx_v.at[1, 0]], s_a, add=True).wait()

    plsc.subcore_barrier()
    pltpu.sync_copy(agg_sh.at[pl.ds(sid * ROWS_PER_S, ROWS_PER_S)],
                    out_hbm.at[cid, pl.ds(sid * ROWS_PER_S, ROWS_PER_S)])


def _sc_aggregate(h, wij, idx_i, idx_j, rcut, zeros):
    idx_pack = jnp.stack([idx_i.reshape(NW, NCHUNK, CHUNK),
                          idx_j.reshape(NW, NCHUNK, CHUNK)], axis=2)
    rc_rep = jnp.repeat(rcut.reshape(NW, NCHUNK, CHUNK // 8, 8), 16, axis=3)
    mesh = plsc.VectorSubcoreMesh(core_axis_name="c", subcore_axis_name="s")
    k = functools.partial(
        pl.kernel,
        mesh=mesh,
        out_type=jax.ShapeDtypeStruct((NC, NPAD, F), jnp.float32),
        scratch_types=[
            pltpu.VMEM((2, 2, CHUNK), jnp.int32),
            pltpu.VMEM((2, CHUNK // 8, 128), jnp.float32),
            pltpu.VMEM((CHUNK, F), jnp.float32),
            pltpu.VMEM((CHUNK, F), jnp.float32),
            pltpu.VMEM((CHUNK, F), jnp.float32),
            pltpu.VMEM((CHUNK, F), jnp.float32),
            pltpu.VMEM_SHARED((NPAD, F), jnp.float32),
            pltpu.SemaphoreType.DMA,
            pltpu.SemaphoreType.DMA,
            pltpu.SemaphoreType.DMA,
            pltpu.SemaphoreType.DMA,
            pltpu.SemaphoreType.DMA,
            pltpu.SemaphoreType.DMA,
        ],
    )(_sc_body)
    return k(h, wij, idx_pack, rc_rep, zeros)


def _out_body(p0_ref, p1_ref, w1_ref, b1_ref, w2_ref, b2_ref, o_ref):
    a = p0_ref[...] + p1_ref[...]
    t = _smu(jnp.dot(a, w1_ref[...], preferred_element_type=jnp.float32)
             + b1_ref[...])
    o_ref[...] = jnp.dot(t, w2_ref[...],
                         preferred_element_type=jnp.float32) + b2_ref[...]


def _compute_out(p0, p1, w1, b1, w2, b2):
    blk = 2000
    return pl.pallas_call(
        _out_body,
        grid=(N // blk,),
        in_specs=[
            pl.BlockSpec((blk, F), lambda i: (i, 0)),
            pl.BlockSpec((blk, F), lambda i: (i, 0)),
            pl.BlockSpec((F, F), lambda i: (0, 0)),
            pl.BlockSpec((1, F), lambda i: (0, 0)),
            pl.BlockSpec((F, F), lambda i: (0, 0)),
            pl.BlockSpec((1, F), lambda i: (0, 0)),
        ],
        out_specs=pl.BlockSpec((blk, F), lambda i: (i, 0)),
        out_shape=jax.ShapeDtypeStruct((N, F), jnp.float32),
    )(p0, p1, w1, b1, w2, b2)


def kernel(x, f_ij, rcut_ij, W_in2f, W_f1, b_f1, W_f2, b_f2,
           W_o1, b_o1, W_o2, b_o2, idx_i, idx_j):
    h = _compute_h(x, W_in2f)
    wij = _compute_wij(f_ij, W_f1, b_f1.reshape(1, F),
                       W_f2, b_f2.reshape(1, F))
    zeros = jnp.zeros((NPAD, F), jnp.float32)
    parts = _sc_aggregate(h, wij, idx_i.astype(jnp.int32),
                          idx_j.astype(jnp.int32), rcut_ij, zeros)
    out = _compute_out(parts[0], parts[1],
                       W_o1, b_o1.reshape(1, F), W_o2, b_o2.reshape(1, F))
    return out


# speedup vs baseline: 1.4250x; 1.4250x over previous
import functools

import jax
import jax.numpy as jnp
from jax import lax
from jax.experimental import pallas as pl
from jax.experimental.pallas import tpu as pltpu
from jax.experimental.pallas import tpu_sc as plsc

N = 10000
E = 320000
F = 128
R = 16

ALPHA = 0.01
MU = 2.5

NC = 2
NS = 16
NW = NC * NS
EPW = E // NW
CHUNK = 80
NCHUNK = EPW // CHUNK
NPAD = 10240
ROWS_PER_S = NPAD // NS


def _smu(x):
    return ((1 + ALPHA) * x
            + (1 - ALPHA) * x * lax.erf(MU * (1 - ALPHA) * x)) / 2


def _h_body(x_ref, w_ref, o_ref):
    o_ref[...] = jnp.dot(x_ref[...], w_ref[...],
                         preferred_element_type=jnp.float32)


def _compute_h(x, w):
    blk = 2000
    return pl.pallas_call(
        _h_body,
        grid=(N // blk,),
        in_specs=[
            pl.BlockSpec((blk, F), lambda i: (i, 0)),
            pl.BlockSpec((F, F), lambda i: (0, 0)),
        ],
        out_specs=pl.BlockSpec((blk, F), lambda i: (i, 0)),
        out_shape=jax.ShapeDtypeStruct((N, F), jnp.float32),
    )(x, w)


def _wij_body(f_ref, rc_ref, w1_ref, b1_ref, w2_ref, b2_ref, o_ref):
    t = jnp.dot(f_ref[...], w1_ref[...], preferred_element_type=jnp.float32)
    t = _smu(t + b1_ref[...])
    w = jnp.dot(t, w2_ref[...], preferred_element_type=jnp.float32)
    o_ref[...] = (w + b2_ref[...]) * rc_ref[...]


def _compute_wij(f_ij, rcut, w1, b1, w2, b2):
    blk = 4000
    return pl.pallas_call(
        _wij_body,
        grid=(E // blk,),
        in_specs=[
            pl.BlockSpec((blk, R), lambda i: (i, 0)),
            pl.BlockSpec((blk, 1), lambda i: (i, 0)),
            pl.BlockSpec((R, F), lambda i: (0, 0)),
            pl.BlockSpec((1, F), lambda i: (0, 0)),
            pl.BlockSpec((F, F), lambda i: (0, 0)),
            pl.BlockSpec((1, F), lambda i: (0, 0)),
        ],
        out_specs=pl.BlockSpec((blk, F), lambda i: (i, 0)),
        out_shape=jax.ShapeDtypeStruct((E, F), jnp.float32),
    )(f_ij, rcut, w1, b1, w2, b2)


def _mul_rows(xj, wij):
    def row_body(r, c2):
        for k in range(F // 16):
            sl = pl.ds(k * 16, 16)
            xj[r, sl] = xj[r, sl] * wij[r, sl]
        return c2
    lax.fori_loop(0, CHUNK, row_body, 0)


def _sc_body(h_hbm, wij_hbm, idx_hbm, zeros_hbm, out_hbm,
             idx_v, xj_a, wij_a, xj_b, wij_b, agg_sh,
             g_a, w_a, s_a, g_b, w_b, s_b):
    cid = lax.axis_index("c")
    sid = lax.axis_index("s")
    wid = sid * NC + cid
    base_e = wid * EPW

    pltpu.sync_copy(zeros_hbm.at[pl.ds(sid * ROWS_PER_S, ROWS_PER_S)],
                    agg_sh.at[pl.ds(sid * ROWS_PER_S, ROWS_PER_S)])
    plsc.subcore_barrier()

    def pair_body(i, carry):
        ta = 2 * i
        tb = 2 * i + 1
        offa = pl.multiple_of(base_e + ta * CHUNK, 16)
        offb = pl.multiple_of(base_e + tb * CHUNK, 16)
        pltpu.sync_copy(idx_hbm.at[wid, pl.ds(ta, 2)], idx_v)
        cga = pltpu.async_copy(h_hbm.at[idx_v.at[0, 1]], xj_a, g_a)
        cwa = pltpu.async_copy(wij_hbm.at[pl.ds(offa, CHUNK)], wij_a, w_a)
        cgb = pltpu.async_copy(h_hbm.at[idx_v.at[1, 1]], xj_b, g_b)
        cwb = pltpu.async_copy(wij_hbm.at[pl.ds(offb, CHUNK)], wij_b, w_b)
        cga.wait()
        cwa.wait()
        _mul_rows(xj_a, wij_a)
        csa = pltpu.async_copy(xj_a, agg_sh.at[idx_v.at[0, 0]], s_a, add=True)
        cgb.wait()
        cwb.wait()
        _mul_rows(xj_b, wij_b)
        csb = pltpu.async_copy(xj_b, agg_sh.at[idx_v.at[1, 0]], s_b, add=True)
        csa.wait()
        csb.wait()
        return carry

    lax.fori_loop(0, NCHUNK // 2, pair_body, 0)

    tt = NCHUNK - 1
    offt = pl.multiple_of(base_e + tt * CHUNK, 16)
    pltpu.sync_copy(idx_hbm.at[wid, pl.ds(tt - 1, 2)], idx_v)
    cgt = pltpu.async_copy(h_hbm.at[idx_v.at[1, 1]], xj_a, g_a)
    cwt = pltpu.async_copy(wij_hbm.at[pl.ds(offt, CHUNK)], wij_a, w_a)
    cgt.wait()
    cwt.wait()
    _mul_rows(xj_a, wij_a)
    pltpu.async_copy(xj_a, agg_sh.at[idx_v.at[1, 0]], s_a, add=True).wait()

    plsc.subcore_barrier()
    pltpu.sync_copy(agg_sh.at[pl.ds(sid * ROWS_PER_S, ROWS_PER_S)],
                    out_hbm.at[cid, pl.ds(sid * ROWS_PER_S, ROWS_PER_S)])


def _sc_aggregate(h, wij, idx_i, idx_j, zeros):
    idx_pair = jnp.stack([idx_i.reshape(NW, NCHUNK, CHUNK),
                          idx_j.reshape(NW, NCHUNK, CHUNK)], axis=2)
    mesh = plsc.VectorSubcoreMesh(core_axis_name="c", subcore_axis_name="s")
    k = functools.partial(
        pl.kernel,
        mesh=mesh,
        out_type=jax.ShapeDtypeStruct((NC, NPAD, F), jnp.float32),
        scratch_types=[
            pltpu.VMEM((2, 2, CHUNK), jnp.int32),
            pltpu.VMEM((CHUNK, F), jnp.float32),
            pltpu.VMEM((CHUNK, F), jnp.float32),
            pltpu.VMEM((CHUNK, F), jnp.float32),
            pltpu.VMEM((CHUNK, F), jnp.float32),
            pltpu.VMEM_SHARED((NPAD, F), jnp.float32),
            pltpu.SemaphoreType.DMA,
            pltpu.SemaphoreType.DMA,
            pltpu.SemaphoreType.DMA,
            pltpu.SemaphoreType.DMA,
            pltpu.SemaphoreType.DMA,
            pltpu.SemaphoreType.DMA,
        ],
    )(_sc_body)
    return k(h, wij, idx_pair, zeros)


def _out_body(p0_ref, p1_ref, w1_ref, b1_ref, w2_ref, b2_ref, o_ref):
    a = p0_ref[...] + p1_ref[...]
    t = _smu(jnp.dot(a, w1_ref[...], preferred_element_type=jnp.float32)
             + b1_ref[...])
    o_ref[...] = jnp.dot(t, w2_ref[...],
                         preferred_element_type=jnp.float32) + b2_ref[...]


def _compute_out(p0, p1, w1, b1, w2, b2):
    blk = 2000
    return pl.pallas_call(
        _out_body,
        grid=(N // blk,),
        in_specs=[
            pl.BlockSpec((blk, F), lambda i: (i, 0)),
            pl.BlockSpec((blk, F), lambda i: (i, 0)),
            pl.BlockSpec((F, F), lambda i: (0, 0)),
            pl.BlockSpec((1, F), lambda i: (0, 0)),
            pl.BlockSpec((F, F), lambda i: (0, 0)),
            pl.BlockSpec((1, F), lambda i: (0, 0)),
        ],
        out_specs=pl.BlockSpec((blk, F), lambda i: (i, 0)),
        out_shape=jax.ShapeDtypeStruct((N, F), jnp.float32),
    )(p0, p1, w1, b1, w2, b2)


def kernel(x, f_ij, rcut_ij, W_in2f, W_f1, b_f1, W_f2, b_f2,
           W_o1, b_o1, W_o2, b_o2, idx_i, idx_j):
    h = _compute_h(x, W_in2f)
    wij = _compute_wij(f_ij, rcut_ij.reshape(E, 1),
                       W_f1, b_f1.reshape(1, F), W_f2, b_f2.reshape(1, F))
    zeros = jnp.zeros((NPAD, F), jnp.float32)
    parts = _sc_aggregate(h, wij, idx_i.astype(jnp.int32),
                          idx_j.astype(jnp.int32), zeros)
    out = _compute_out(parts[0], parts[1],
                       W_o1, b_o1.reshape(1, F), W_o2, b_o2.reshape(1, F))
    return out
